# baseline (speedup 1.0000x reference)
import functools
import jax
import jax.numpy as jnp
from jax import lax
from jax.experimental import pallas as pl
from jax.experimental.pallas import tpu as pltpu

B, S, W, H = 4, 2500, 128, 128
P = W * H
N = B * S
E = 160000
C = 256

S_PAD = 2560
PK = 512
K_STEPS = P // PK


def _pool_body(lab_ref, s0_ref, s1_ref, s2_ref, sm_ref,
               o0_ref, o1_ref, o2_ref, om_ref,
               a0, a1, a2, am):
    k = pl.program_id(2)
    s_tile = pl.program_id(1)

    lab = lab_ref[0, 0, :]
    s_ids = s_tile * 128 + lax.broadcasted_iota(jnp.int32, (PK, 128), 1)
    eq = jnp.where(lab[:, None] == s_ids, 1.0, 0.0).astype(jnp.bfloat16)

    @pl.when(k == 0)
    def _():
        a0[...] = jnp.zeros_like(a0)
        a1[...] = jnp.zeros_like(a1)
        a2[...] = jnp.zeros_like(a2)
        am[...] = jnp.zeros_like(am)

    def acc(a_ref, f_ref):
        f = f_ref[0].astype(jnp.bfloat16)
        a_ref[...] += lax.dot_general(
            eq, f, (((0,), (1,)), ((), ())),
            preferred_element_type=jnp.float32)

    acc(a0, s0_ref)
    acc(a1, s1_ref)
    acc(a2, s2_ref)
    acc(am, sm_ref)

    @pl.when(k == K_STEPS - 1)
    def _():
        o0_ref[0] = a0[...]
        o1_ref[0] = a1[...]
        o2_ref[0] = a2[...]
        om_ref[0] = am[...]


def _pool_all(labels, s0, s1, s2, small):
    grid = (B, S_PAD // 128, K_STEPS)
    labels = labels.reshape(B * K_STEPS, 1, PK)
    lab_spec = pl.BlockSpec((1, 1, PK), lambda b, s, k: (b * K_STEPS + k, 0, 0))
    f_spec = pl.BlockSpec((1, C, PK), lambda b, s, k: (b, 0, k))
    sm_spec = pl.BlockSpec((1, 8, PK), lambda b, s, k: (b, 0, k))
    o_spec = pl.BlockSpec((1, 128, C), lambda b, s, k: (b, s, 0))
    om_spec = pl.BlockSpec((1, 128, 8), lambda b, s, k: (b, s, 0))
    out_shapes = [
        jax.ShapeDtypeStruct((B, S_PAD, C), jnp.float32),
        jax.ShapeDtypeStruct((B, S_PAD, C), jnp.float32),
        jax.ShapeDtypeStruct((B, S_PAD, C), jnp.float32),
        jax.ShapeDtypeStruct((B, S_PAD, 8), jnp.float32),
    ]
    return pl.pallas_call(
        _pool_body,
        grid=grid,
        in_specs=[lab_spec, f_spec, f_spec, f_spec, sm_spec],
        out_specs=[o_spec, o_spec, o_spec, om_spec],
        out_shape=out_shapes,
        scratch_shapes=[
            pltpu.VMEM((128, C), jnp.float32),
            pltpu.VMEM((128, C), jnp.float32),
            pltpu.VMEM((128, C), jnp.float32),
            pltpu.VMEM((128, 8), jnp.float32),
        ],
    )(labels, s0, s1, s2, small)


def _bn(x, g, b, eps=1e-5):
    mu = x.mean(axis=0)
    var = x.var(axis=0)
    return (x - mu) / jnp.sqrt(var + eps) * g + b


def _gcn(x, src, dst, ew, Wt, bias):
    n = x.shape[0]
    xw = x @ Wt
    loop = jnp.arange(n)
    s = jnp.concatenate([src, loop])
    d = jnp.concatenate([dst, loop])
    w = jnp.concatenate([ew, jnp.ones((n,), x.dtype)])
    deg = jnp.zeros((n,), x.dtype).at[d].add(w)
    dinv = jnp.where(deg > 0, 1.0 / jnp.sqrt(deg), 0.0)
    norm = dinv[s] * w * dinv[d]
    out = jnp.zeros_like(xw).at[d].add(norm[:, None] * xw[s])
    return out + bias


def kernel(fx, fy, skip0, skip1, skip2, params, labels, edges_nn):
    lab = labels.reshape(B, P)
    s0 = skip0.reshape(B, C, P)
    s1 = skip1.reshape(B, C, P)
    s2 = skip2.reshape(B, C, P)

    xx = jnp.broadcast_to(jnp.arange(W, dtype=jnp.float32)[:, None], (W, H)) / (W - 1)
    yy = jnp.broadcast_to(jnp.arange(H, dtype=jnp.float32)[None, :], (W, H)) / (H - 1)
    cm = jnp.broadcast_to(jnp.stack([xx, yy], axis=0)[None], (B, 2, W, H))
    small = jnp.concatenate(
        [cm.reshape(B, 2, P), fx.reshape(B, 1, P), fy.reshape(B, 1, P),
         jnp.ones((B, 1, P), jnp.float32), jnp.zeros((B, 3, P), jnp.float32)],
        axis=1)

    p0, p1, p2, pm = _pool_all(lab, s0, s1, s2, small)
    p0 = p0[:, :S].reshape(N, C)
    p1 = p1[:, :S].reshape(N, C)
    p2 = p2[:, :S].reshape(N, C)
    pm = pm[:, :S].reshape(N, 8)
    cnt = jnp.maximum(pm[:, 4], 1.0)[:, None]
    inv = 1.0 / cnt

    coords = jnp.concatenate([pm[:, 0:2] * inv, pm[:, 2:3] * inv, pm[:, 3:4] * inv], axis=1)
    coords = jax.nn.relu(_bn(coords, params['pre_g'], params['pre_b']))

    src = edges_nn[0]
    dst = edges_nn[1]
    ew = jnp.where(edges_nn[2] != -1, 1.0, 0.0).astype(jnp.float32)

    pooled = [p0 * inv, p1 * inv, p2 * inv]
    x = None
    for i in range(3):
        skip = pooled[i]
        if i == 0:
            x = skip
        else:
            x = jnp.concatenate([x, skip], axis=1)
            x = _bn(x @ params['mW%d' % (i - 1)] + params['mb%d' % (i - 1)],
                    params['mbn_g%d' % (i - 1)], params['mbn_b%d' % (i - 1)])
        x = _gcn(x, src, dst, ew, params['gW%d' % i], params['gb%d' % i])
        x = jax.nn.relu(_bn(x, params['gbn_g%d' % i], params['gbn_b%d' % i]))
    x = jnp.concatenate([x, coords], axis=1)
    x = _bn(x @ params['mW2'] + params['mb2'], params['mbn_g2'], params['mbn_b2'])
    x = jax.nn.relu(x @ params['lW'] + params['lb'])
    return x


# speedup vs baseline: 2.0596x; 2.0596x over previous
import functools
import jax
import jax.numpy as jnp
from jax import lax
from jax.experimental import pallas as pl
from jax.experimental.pallas import tpu as pltpu
from jax.experimental.pallas import tpu_sc as plsc

B, S, W, H = 4, 2500, 128, 128
P = W * H
N = B * S
E = 160000
C = 256

S_PAD = 2560
PK = 512
K_STEPS = P // PK

N2 = 2 * N
ZROW = N2
YROWS = N2 + 8
NSUB = 16
WAVE = 2
KCH = 80
EPT = KCH * 128
E_PAD = NSUB * EPT
NHALF = N // 2
AROWS = 5120
TRASH = AROWS - 1
NROW_T = AROWS // NSUB


def _pool_body(lab_ref, s0_ref, s1_ref, s2_ref, sm_ref,
               o0_ref, o1_ref, o2_ref, om_ref,
               a0, a1, a2, am):
    k = pl.program_id(2)
    s_tile = pl.program_id(1)

    lab = lab_ref[0, 0, :]
    s_ids = s_tile * 128 + lax.broadcasted_iota(jnp.int32, (PK, 128), 1)
    eq = jnp.where(lab[:, None] == s_ids, 1.0, 0.0).astype(jnp.bfloat16)

    @pl.when(k == 0)
    def _():
        a0[...] = jnp.zeros_like(a0)
        a1[...] = jnp.zeros_like(a1)
        a2[...] = jnp.zeros_like(a2)
        am[...] = jnp.zeros_like(am)

    def acc(a_ref, f_ref):
        f = f_ref[0].astype(jnp.bfloat16)
        a_ref[...] += lax.dot_general(
            eq, f, (((0,), (1,)), ((), ())),
            preferred_element_type=jnp.float32)

    acc(a0, s0_ref)
    acc(a1, s1_ref)
    acc(a2, s2_ref)
    acc(am, sm_ref)

    @pl.when(k == K_STEPS - 1)
    def _():
        o0_ref[0] = a0[...]
        o1_ref[0] = a1[...]
        o2_ref[0] = a2[...]
        om_ref[0] = am[...]


def _pool_all(labels, s0, s1, s2, small):
    grid = (B, S_PAD // 128, K_STEPS)
    labels = labels.reshape(B * K_STEPS, 1, PK)
    lab_spec = pl.BlockSpec((1, 1, PK), lambda b, s, k: (b * K_STEPS + k, 0, 0))
    f_spec = pl.BlockSpec((1, C, PK), lambda b, s, k: (b, 0, k))
    sm_spec = pl.BlockSpec((1, 8, PK), lambda b, s, k: (b, 0, k))
    o_spec = pl.BlockSpec((1, 128, C), lambda b, s, k: (b, s, 0))
    om_spec = pl.BlockSpec((1, 128, 8), lambda b, s, k: (b, s, 0))
    out_shapes = [
        jax.ShapeDtypeStruct((B, S_PAD, C), jnp.float32),
        jax.ShapeDtypeStruct((B, S_PAD, C), jnp.float32),
        jax.ShapeDtypeStruct((B, S_PAD, C), jnp.float32),
        jax.ShapeDtypeStruct((B, S_PAD, 8), jnp.float32),
    ]
    return pl.pallas_call(
        _pool_body,
        grid=grid,
        in_specs=[lab_spec, f_spec, f_spec, f_spec, sm_spec],
        out_specs=[o_spec, o_spec, o_spec, om_spec],
        out_shape=out_shapes,
        scratch_shapes=[
            pltpu.VMEM((128, C), jnp.float32),
            pltpu.VMEM((128, C), jnp.float32),
            pltpu.VMEM((128, C), jnp.float32),
            pltpu.VMEM((128, 8), jnp.float32),
        ],
    )(labels, s0, s1, s2, small)


def _sc_agg_body(ypad_hbm, srcg_hbm, dstg_hbm, out_hbm,
                 sidx, didx, ring, acc, gs0, gs1, ss0, ss1):
    c = lax.axis_index("c")
    sid = lax.axis_index("s")
    gsems = (gs0, gs1)
    ssems = (ss0, ss1)
    n_waves = KCH // WAVE

    pltpu.sync_copy(srcg_hbm.at[c, sid], sidx)

    def gather_wave(w, g):
        for b in range(WAVE):
            pltpu.async_copy(ypad_hbm.at[sidx.at[w * WAVE + b]],
                             ring.at[g, b], gsems[g])

    def wait_gathers(g):
        for b in range(WAVE):
            pltpu.make_async_copy(ypad_hbm.at[pl.ds(0, 128)],
                                  ring.at[g, b], gsems[g]).wait()

    def scatter_wave(w, g):
        for b in range(WAVE):
            pltpu.async_copy(ring.at[g, b], acc.at[didx.at[w * WAVE + b]],
                             ssems[g], add=True)

    def wait_scatters(g):
        for b in range(WAVE):
            pltpu.make_async_copy(ypad_hbm.at[pl.ds(0, 128)],
                                  ring.at[g, b], ssems[g]).wait()

    def step_body(w, g):
        @pl.when(w + 1 < n_waves)
        def _():
            @pl.when(w >= 1)
            def _():
                wait_scatters(1 - g)
            gather_wave(w + 1, 1 - g)

        wait_gathers(g)
        scatter_wave(w, g)

    def step_pair(wp, _):
        step_body(2 * wp, 0)
        step_body(2 * wp + 1, 1)
        return 0

    for p in range(2):
        pltpu.sync_copy(dstg_hbm.at[p, sid], didx)

        def zrow(r, _):
            for j in range(8):
                ring[0, 0, r, pl.ds(j * 16, 16)] = jnp.zeros((16,), jnp.float32)
            return 0
        lax.fori_loop(0, 128, zrow, 0)
        for j in range(3):
            nr = 128 if j < 2 else NROW_T - 256
            pltpu.sync_copy(ring.at[0, 0, pl.ds(0, nr)],
                            acc.at[pl.ds(sid * NROW_T + j * 128, nr)])
        plsc.subcore_barrier()

        gather_wave(0, 0)
        lax.fori_loop(0, n_waves // 2, step_pair, 0, unroll=False)
        wait_scatters(0)
        wait_scatters(1)
        plsc.subcore_barrier()

        pltpu.sync_copy(acc.at[pl.ds(sid * NROW_T, NROW_T)],
                        out_hbm.at[p, pl.ds(sid * NROW_T, NROW_T), c])
        plsc.subcore_barrier()


def _sc_agg(ypad, srcg, dstg):
    mesh = plsc.VectorSubcoreMesh(core_axis_name="c", subcore_axis_name="s")
    return pl.kernel(
        _sc_agg_body,
        out_type=jax.ShapeDtypeStruct((2, AROWS, 2, 128), jnp.float32),
        mesh=mesh,
        scratch_types=[
            pltpu.VMEM((KCH, 128), jnp.int32),
            pltpu.VMEM((KCH, 128), jnp.int32),
            pltpu.VMEM((2, WAVE, 128, 128), jnp.float32),
            pltpu.VMEM_SHARED((AROWS, 128), jnp.float32),
            pltpu.SemaphoreType.DMA,
            pltpu.SemaphoreType.DMA,
            pltpu.SemaphoreType.DMA,
            pltpu.SemaphoreType.DMA,
        ],
    )(ypad, srcg, dstg)


def _bn(x, g, b, eps=1e-5):
    mu = x.mean(axis=0)
    var = x.var(axis=0)
    return (x - mu) / jnp.sqrt(var + eps) * g + b


def _gcn_sc(x, dinv, srcg, dstg, Wt, bias):
    xw = x @ Wt
    y = dinv[:, None] * xw
    ypad = jnp.concatenate([y.reshape(N2, 128),
                            jnp.zeros((8, 128), jnp.float32)], axis=0)
    agg = _sc_agg(ypad, srcg, dstg)[:, :NHALF].reshape(N, C)
    return dinv[:, None] * (agg + y) + bias


def kernel(fx, fy, skip0, skip1, skip2, params, labels, edges_nn):
    lab = labels.reshape(B, P)
    s0 = skip0.reshape(B, C, P)
    s1 = skip1.reshape(B, C, P)
    s2 = skip2.reshape(B, C, P)

    xx = jnp.broadcast_to(jnp.arange(W, dtype=jnp.float32)[:, None], (W, H)) / (W - 1)
    yy = jnp.broadcast_to(jnp.arange(H, dtype=jnp.float32)[None, :], (W, H)) / (H - 1)
    cm = jnp.broadcast_to(jnp.stack([xx, yy], axis=0)[None], (B, 2, W, H))
    small = jnp.concatenate(
        [cm.reshape(B, 2, P), fx.reshape(B, 1, P), fy.reshape(B, 1, P),
         jnp.ones((B, 1, P), jnp.float32), jnp.zeros((B, 3, P), jnp.float32)],
        axis=1)

    p0, p1, p2, pm = _pool_all(lab, s0, s1, s2, small)
    p0 = p0[:, :S].reshape(N, C)
    p1 = p1[:, :S].reshape(N, C)
    p2 = p2[:, :S].reshape(N, C)
    pm = pm[:, :S].reshape(N, 8)
    cnt = jnp.maximum(pm[:, 4], 1.0)[:, None]
    inv = 1.0 / cnt

    coords = jnp.concatenate([pm[:, 0:2] * inv, pm[:, 2:3] * inv, pm[:, 3:4] * inv], axis=1)
    coords = jax.nn.relu(_bn(coords, params['pre_g'], params['pre_b']))

    src = edges_nn[0]
    dst = edges_nn[1]
    ewb = edges_nn[2] != -1
    ew = jnp.where(ewb, 1.0, 0.0).astype(jnp.float32)

    deg = jnp.zeros((N,), jnp.float32).at[dst].add(ew) + 1.0
    dinv = lax.rsqrt(deg)

    npad = E_PAD - E
    src_m = jnp.where(ewb, 2 * src.astype(jnp.int32), ZROW)
    src_p = jnp.concatenate([src_m, jnp.full((npad,), ZROW, jnp.int32)])
    srcg = jnp.stack([src_p, src_p + 1]).reshape(2, NSUB, KCH, 128)
    dst_p = jnp.concatenate([dst.astype(jnp.int32),
                             jnp.full((npad,), TRASH, jnp.int32)])
    dst_passes = []
    for p in range(2):
        lo = p * NHALF
        inr = (dst_p >= lo) & (dst_p < lo + NHALF)
        dst_passes.append(jnp.where(inr, dst_p - lo, TRASH))
    dstg = jnp.stack(dst_passes).reshape(2, NSUB, KCH, 128)

    pooled = [p0 * inv, p1 * inv, p2 * inv]
    x = None
    for i in range(3):
        skip = pooled[i]
        if i == 0:
            x = skip
        else:
            x = jnp.concatenate([x, skip], axis=1)
            x = _bn(x @ params['mW%d' % (i - 1)] + params['mb%d' % (i - 1)],
                    params['mbn_g%d' % (i - 1)], params['mbn_b%d' % (i - 1)])
        x = _gcn_sc(x, dinv, srcg, dstg, params['gW%d' % i], params['gb%d' % i])
        x = jax.nn.relu(_bn(x, params['gbn_g%d' % i], params['gbn_b%d' % i]))
    x = jnp.concatenate([x, coords], axis=1)
    x = _bn(x @ params['mW2'] + params['mb2'], params['mbn_g2'], params['mbn_b2'])
    x = jax.nn.relu(x @ params['lW'] + params['lb'])
    return x
